# baseline (speedup 1.0000x reference)
import functools

import jax
import jax.numpy as jnp
from jax import lax
from jax.experimental import pallas as pl
from jax.experimental.pallas import tpu as pltpu
from jax.experimental.pallas import tpu_sc as plsc

NSEG = 512
D = 128
NHEADS = 4
BLK = 2048
ROWW = 640
AUXW = 16
NW = 32
SEG_PER_TILE = NSEG // NW
CH = 128


def _gate_body(x_ref, w1_ref, b1_ref, w2_ref, b2_ref, e_ref, g_scr, k_scr):
    p = pl.program_id(0)
    i = pl.program_id(1)

    @pl.when(p == 0)
    def _phase0():
        xb = x_ref[...]
        h = jnp.maximum(
            jnp.dot(xb, w1_ref[...], preferred_element_type=jnp.float32)
            + b1_ref[...], 0.0)
        g = (jnp.dot(h, w2_ref[...], preferred_element_type=jnp.float32)
             + b2_ref[...])
        g_scr[i] = g
        bmax = jnp.max(g)
        prev = jnp.where(i == 0, -jnp.inf, k_scr[0])
        k_scr[0] = jnp.maximum(prev, bmax)

    @pl.when(p == 1)
    def _phase1():
        e_ref[...] = jnp.exp(g_scr[i] - k_scr[0])


def _sc_body(x_hbm, aux_hbm, starts_hbm, out_hbm,
             xbuf, auxbuf, stbuf, acc):
    c = lax.axis_index("c")
    s = lax.axis_index("s")
    wid = s * 2 + c
    lo = wid * SEG_PER_TILE
    hi = lo + SEG_PER_TILE

    def _z(t, _):
        acc[t // (ROWW // 16), pl.ds((t % (ROWW // 16)) * 16, 16)] = (
            jnp.zeros((16,), jnp.float32))
        return 0
    lax.fori_loop(0, SEG_PER_TILE * (ROWW // 16), _z, 0)

    pltpu.sync_copy(starts_hbm, stbuf)
    r0 = stbuf[pl.ds(wid, 16)][0]
    r1 = stbuf[pl.ds(wid + 1, 16)][0]
    a0 = (r0 // CH) * CH
    nch = (r1 - a0 + CH - 1) // CH

    def _group(lrow):
        av0 = auxbuf[lrow, pl.ds(0, 16)]
        av15 = auxbuf[lrow + 15, pl.ds(0, 16)]
        sg0 = av0[NHEADS].astype(jnp.int32)
        sg15 = av15[NHEADS].astype(jnp.int32)
        fast = (sg0 == sg15) & (sg0 >= lo) & (sg0 < hi)

        @pl.when(fast)
        def _fast():
            l = sg0 - lo
            accs = [acc[l, pl.ds(k * 16, 16)] for k in range(8 * NHEADS)]
            sep = acc[l, pl.ds(512, 16)]
            for j in range(16):
                av = auxbuf[lrow + j, pl.ds(0, 16)]
                sep = sep + av
                for hh in range(NHEADS):
                    eb = jnp.full((16,), av[hh], jnp.float32)
                    for k in range(8):
                        accs[hh * 8 + k] = (accs[hh * 8 + k]
                                            + eb * xbuf[lrow + j,
                                                        pl.ds(k * 16, 16)])
            for k in range(8 * NHEADS):
                acc[l, pl.ds(k * 16, 16)] = accs[k]
            acc[l, pl.ds(512, 16)] = sep

        @pl.when(jnp.logical_not(fast))
        def _slow():
            def _row(j, _):
                av = auxbuf[lrow + j, pl.ds(0, 16)]
                sg = av[NHEADS].astype(jnp.int32)

                @pl.when((sg >= lo) & (sg < hi))
                def _():
                    l = sg - lo
                    sep = acc[l, pl.ds(512, 16)]
                    acc[l, pl.ds(512, 16)] = sep + av
                    for hh in range(NHEADS):
                        eb = jnp.full((16,), av[hh], jnp.float32)
                        for k in range(8):
                            a = acc[l, pl.ds(hh * 128 + k * 16, 16)]
                            acc[l, pl.ds(hh * 128 + k * 16, 16)] = (
                                a + eb * xbuf[lrow + j, pl.ds(k * 16, 16)])
                return 0
            lax.fori_loop(0, 16, _row, 0)

    def _chunk(ci, _):
        rbase = a0 + ci * CH
        pltpu.sync_copy(x_hbm.at[pl.ds(rbase, CH)], xbuf)
        pltpu.sync_copy(aux_hbm.at[pl.ds(rbase, CH)], auxbuf)
        for g in range(CH // 16):
            _group(g * 16)
        return 0

    lax.fori_loop(0, nch, _chunk, 0)

    pltpu.sync_copy(acc, out_hbm.at[pl.ds(lo, SEG_PER_TILE)])


def _combine_body(p_ref, o_ref):
    tot = p_ref[...]
    for hh in range(NHEADS):
        o_ref[:, hh * D:(hh + 1) * D] = (
            tot[:, hh * D:(hh + 1) * D]
            / (tot[:, 512 + hh:513 + hh] + 1e-16))


def kernel(x, batch, W1_0, b1_0, W2_0, b2_0, W1_1, b1_1, W2_1, b2_1,
           W1_2, b1_2, W2_2, b2_2, W1_3, b1_3, W2_3, b2_3):
    n = x.shape[0]
    nblk = -(-n // BLK)
    npad = nblk * BLK
    x16 = jnp.pad(x, ((0, npad - n), (0, 0))).astype(jnp.bfloat16)

    w1t = jnp.concatenate([W1_0.T, W1_1.T, W1_2.T, W1_3.T],
                          axis=1).astype(jnp.bfloat16)
    b1c = jnp.concatenate([b1_0, b1_1, b1_2, b1_3]).reshape(1, 4 * 128)
    w2blk = jnp.zeros((4 * 128, NHEADS), jnp.float32)
    for hh, w2 in enumerate([W2_0, W2_1, W2_2, W2_3]):
        w2blk = w2blk.at[hh * 128:(hh + 1) * 128, hh].set(w2[0])
    b2c = jnp.stack([b2_0[0], b2_1[0], b2_2[0], b2_3[0]]).reshape(1, NHEADS)

    ew = pl.pallas_call(
        _gate_body,
        grid=(2, nblk),
        in_specs=[
            pl.BlockSpec((BLK, D), lambda p, i: (i * (1 - p), 0)),
            pl.BlockSpec((D, 4 * 128), lambda p, i: (0, 0)),
            pl.BlockSpec((1, 4 * 128), lambda p, i: (0, 0)),
            pl.BlockSpec((4 * 128, NHEADS), lambda p, i: (0, 0)),
            pl.BlockSpec((1, NHEADS), lambda p, i: (0, 0)),
        ],
        out_specs=pl.BlockSpec((BLK, NHEADS), lambda p, i: (i, 0)),
        out_shape=jax.ShapeDtypeStruct((npad, NHEADS), jnp.float32),
        scratch_shapes=[
            pltpu.VMEM((nblk, BLK, NHEADS), jnp.float32),
            pltpu.SMEM((1,), jnp.float32),
        ],
        compiler_params=pltpu.CompilerParams(
            dimension_semantics=("arbitrary", "arbitrary")),
    )(x16, w1t, b1c, w2blk, b2c)

    npc = -(-n // CH) * CH
    aux = jnp.zeros((npc, AUXW), jnp.float32)
    aux = aux.at[:n, :NHEADS].set(ew[:n])
    aux = aux.at[:n, NHEADS].set(batch.astype(jnp.float32))
    starts = jnp.searchsorted(
        batch, jnp.arange(0, NSEG + SEG_PER_TILE, SEG_PER_TILE,
                          dtype=jnp.int32)).astype(jnp.int32)
    starts = jnp.pad(starts, (0, 48 - starts.shape[0]))

    mesh = plsc.VectorSubcoreMesh(core_axis_name="c", subcore_axis_name="s")
    part = pl.kernel(
        _sc_body,
        mesh=mesh,
        out_type=jax.ShapeDtypeStruct((NSEG, ROWW), jnp.float32),
        scratch_types=[
            pltpu.VMEM((CH, D), jnp.float32),
            pltpu.VMEM((CH, AUXW), jnp.float32),
            pltpu.VMEM((48,), jnp.int32),
            pltpu.VMEM((SEG_PER_TILE, ROWW), jnp.float32),
        ],
    )(jnp.pad(x, ((0, npc - n), (0, 0))), aux, starts)

    out = pl.pallas_call(
        _combine_body,
        in_specs=[pl.BlockSpec((NSEG, ROWW), lambda: (0, 0))],
        out_specs=pl.BlockSpec((NSEG, NHEADS * D), lambda: (0, 0)),
        out_shape=jax.ShapeDtypeStruct((NSEG, NHEADS * D), jnp.float32),
    )(part)
    return out


# speedup vs baseline: 4.1585x; 4.1585x over previous
import functools

import jax
import jax.numpy as jnp
from jax import lax
from jax.experimental import pallas as pl
from jax.experimental.pallas import tpu as pltpu

NSEG = 512
D = 128
NHEADS = 4
BLK = 2048
LSPAN = 64


def _fused_body(batch_ref, x_ref, w1_ref, b1_ref, w2_ref, b2_ref,
                out_ref, g_scr, se_scr, k_scr):
    p = pl.program_id(0)
    i = pl.program_id(1)
    nblk = pl.num_programs(1)

    @pl.when(p == 0)
    def _phase0():
        xb = x_ref[...]
        h = jnp.maximum(
            jnp.dot(xb, w1_ref[...], preferred_element_type=jnp.float32)
            + b1_ref[...], 0.0)
        g = (jnp.dot(h, w2_ref[...], preferred_element_type=jnp.float32)
             + b2_ref[...])
        g_scr[i] = g
        bmax = jnp.max(g)
        prev = jnp.where(i == 0, -jnp.inf, k_scr[0])
        k_scr[0] = jnp.maximum(prev, bmax)

    @pl.when(p == 1)
    def _phase1():
        @pl.when(i == 0)
        def _init():
            out_ref[...] = jnp.zeros_like(out_ref)
            se_scr[...] = jnp.zeros_like(se_scr)

        g = g_scr[i]
        e = jnp.exp(g - k_scr[0]).astype(jnp.bfloat16)
        seg = batch_ref[0]
        xb = x_ref[...]
        ex = jnp.concatenate(
            [e[:, hh:hh + 1] * xb for hh in range(NHEADS)], axis=1)

        s0 = seg[0, 0]
        s0a = jnp.minimum((s0 // 8) * 8, NSEG - LSPAN)
        smax = seg[0, BLK - 1]
        fits = (smax - s0a) < LSPAN

        @pl.when(fits)
        def _local():
            oh = (lax.broadcasted_iota(jnp.int32, (LSPAN, BLK), 0) + s0a
                  == seg).astype(jnp.bfloat16)
            se_scr[pl.ds(s0a, LSPAN), :] += jnp.dot(
                oh, e, preferred_element_type=jnp.float32)
            out_ref[pl.ds(s0a, LSPAN), :] += jnp.dot(
                oh, ex, preferred_element_type=jnp.float32)

        @pl.when(jnp.logical_not(fits))
        def _full():
            oh = (lax.broadcasted_iota(jnp.int32, (NSEG, BLK), 0)
                  == seg).astype(jnp.bfloat16)
            se_scr[...] += jnp.dot(oh, e, preferred_element_type=jnp.float32)
            out_ref[...] += jnp.dot(oh, ex, preferred_element_type=jnp.float32)

    @pl.when((p == 1) & (i == nblk - 1))
    def _finish():
        se = se_scr[...]
        for hh in range(NHEADS):
            out_ref[:, hh * D:(hh + 1) * D] = (
                out_ref[:, hh * D:(hh + 1) * D]
                / (se[:, hh:hh + 1] + 1e-16))


@functools.partial(jax.jit, static_argnames=())
def kernel(x, batch, W1_0, b1_0, W2_0, b2_0, W1_1, b1_1, W2_1, b2_1,
           W1_2, b1_2, W2_2, b2_2, W1_3, b1_3, W2_3, b2_3):
    n = x.shape[0]
    nblk = -(-n // BLK)
    npad = nblk * BLK
    x_p = jnp.pad(x, ((0, npad - n), (0, 0))).astype(jnp.bfloat16)
    batch_p = jnp.pad(batch, (0, npad - n), constant_values=NSEG)
    batch_p = batch_p.reshape(nblk, 1, BLK)

    w1t = jnp.concatenate([W1_0.T, W1_1.T, W1_2.T, W1_3.T], axis=1).astype(jnp.bfloat16)
    b1c = jnp.concatenate([b1_0, b1_1, b1_2, b1_3]).reshape(1, 4 * 128)
    w2blk = jnp.zeros((4 * 128, NHEADS), jnp.float32)
    for hh, w2 in enumerate([W2_0, W2_1, W2_2, W2_3]):
        w2blk = w2blk.at[hh * 128:(hh + 1) * 128, hh].set(w2[0])
    b2c = jnp.stack([b2_0[0], b2_1[0], b2_2[0], b2_3[0]]).reshape(1, NHEADS)

    out = pl.pallas_call(
        _fused_body,
        grid=(2, nblk),
        in_specs=[
            pl.BlockSpec((1, 1, BLK), lambda p, i: (i, 0, 0)),
            pl.BlockSpec((BLK, D), lambda p, i: (i, 0)),
            pl.BlockSpec((D, 4 * 128), lambda p, i: (0, 0)),
            pl.BlockSpec((1, 4 * 128), lambda p, i: (0, 0)),
            pl.BlockSpec((4 * 128, NHEADS), lambda p, i: (0, 0)),
            pl.BlockSpec((1, NHEADS), lambda p, i: (0, 0)),
        ],
        out_specs=pl.BlockSpec((NSEG, NHEADS * D), lambda p, i: (0, 0)),
        out_shape=jax.ShapeDtypeStruct((NSEG, NHEADS * D), jnp.float32),
        scratch_shapes=[
            pltpu.VMEM((nblk, BLK, NHEADS), jnp.float32),
            pltpu.VMEM((NSEG, NHEADS), jnp.float32),
            pltpu.SMEM((1,), jnp.float32),
        ],
        compiler_params=pltpu.CompilerParams(
            dimension_semantics=("arbitrary", "arbitrary")),
    )(batch_p, x_p, w1t, b1c, w2blk, b2c)
    return out
